# baseline (speedup 1.0000x reference)
import functools

import jax
import jax.numpy as jnp
from jax import lax
from jax.experimental import pallas as pl
from jax.experimental.pallas import tpu as pltpu
from jax.experimental.pallas import tpu_sc as plsc

N = 10000
E = 320000
D_IN = 128
D_HID = 16

NC = 2
NS = 16
NW = NC * NS
CHUNK = 128
NCH = 80
Q = NCH * CHUNK
EPAD = NW * Q
NPAD = 10240
RPS = NPAD // NS
SUP = 16
NSUP = NCH // SUP

_mesh = plsc.VectorSubcoreMesh(core_axis_name="c", subcore_axis_name="s")
_sc_params = pltpu.CompilerParams(use_tc_tiling_on_sc=False,
                                  needs_layout_passes=False)


DROWS = 640


@functools.partial(
    pl.kernel,
    out_type=jax.ShapeDtypeStruct((NC, NPAD, 16), jnp.float32),
    mesh=_mesh,
    scratch_types=[
        pltpu.VMEM((NSUP, SUP * CHUNK), jnp.int32),
        pltpu.VMEM((DROWS, 16), jnp.float32),
        pltpu.VMEM((1, DROWS), jnp.int32),
        pltpu.VMEM((DROWS // NS, 16), jnp.float32),
        pltpu.VMEM((RPS, 16), jnp.float32),
        pltpu.VMEM_SHARED((DROWS, 16), jnp.float32),
        pltpu.SemaphoreType.DMA,
    ],
    compiler_params=_sc_params,
)
def _sc_degree(dst_hbm, zeros_hbm, out_hbm, idx_v, hist_v, iota_v, pk_v,
               bc_v, acc_sh, dsem):
    c = lax.axis_index("c")
    s = lax.axis_index("s")
    wid = c * NS + s
    di = pltpu.async_copy(dst_hbm.at[pl.ds(wid * NSUP, NSUP)], idx_v, dsem)
    zvec = jnp.zeros((16,), jnp.float32)

    @pl.loop(0, DROWS)
    def _(r):
        hist_v[r, :] = zvec

    @pl.loop(0, DROWS // 16)
    def _(m):
        iota_v[0, pl.ds(m * 16, 16)] = lax.iota(jnp.int32, 16) + m * 16

    @pl.when(s == 0)
    def _():
        pltpu.sync_copy(zeros_hbm.at[pl.ds(0, DROWS)], acc_sh)

    ones_vec = jnp.ones((16,), jnp.float32)
    di.wait()

    @pl.loop(0, NSUP)
    def _(j):
        for k in range(SUP * CHUNK // 16):
            d = idx_v[j, pl.ds(k * 16, 16)]
            plsc.addupdate_scatter(hist_v, [d >> 4, d & 15], ones_vec)

    plsc.subcore_barrier()
    pltpu.async_copy(hist_v, acc_sh.at[iota_v.at[0]], dsem, add=True).wait()
    plsc.subcore_barrier()
    pltpu.sync_copy(acc_sh.at[pl.ds(s * (DROWS // NS), DROWS // NS)], pk_v)

    @pl.loop(0, RPS)
    def _(i):
        bc_v[i, :] = plsc.load_gather(
            pk_v, [jnp.full((16,), i >> 4, jnp.int32),
                   jnp.full((16,), i & 15, jnp.int32)])

    pltpu.sync_copy(bc_v, out_hbm.at[c].at[pl.ds(s * RPS, RPS)])


@functools.partial(
    pl.kernel,
    out_type=jax.ShapeDtypeStruct((NC, NPAD, D_HID), jnp.float32),
    mesh=_mesh,
    scratch_types=[
        pltpu.VMEM((NSUP, SUP * CHUNK), jnp.int32),
        pltpu.VMEM((NSUP, SUP * CHUNK), jnp.int32),
        [pltpu.VMEM((SUP * CHUNK, D_HID), jnp.float32) for _ in range(2)],
        pltpu.VMEM_SHARED((NPAD, D_HID), jnp.float32),
        pltpu.VMEM_SHARED((NPAD, D_HID), jnp.float32),
        [pltpu.SemaphoreType.DMA for _ in range(4)],
        [pltpu.SemaphoreType.DMA for _ in range(4)],
        pltpu.SemaphoreType.DMA,
        pltpu.SemaphoreType.DMA,
        pltpu.SemaphoreType.DMA,
    ],
    compiler_params=_sc_params,
)
def _sc_edge_pass(g_hbm, src_hbm, dst_hbm, zeros_hbm, out_hbm,
                  src_v, dst_v, rows_v, acc_sh, g_sh, gsems, ssems,
                  isem, tsem, asem):
    c = lax.axis_index("c")
    s = lax.axis_index("s")
    wid = c * NS + s
    s_off = s * RPS
    tail = N - (NS - 1) * RPS
    di1 = pltpu.async_copy(src_hbm.at[pl.ds(wid * NSUP, NSUP)], src_v, isem)
    di2 = pltpu.async_copy(dst_hbm.at[pl.ds(wid * NSUP, NSUP)], dst_v, isem)

    @pl.when(s < NS - 1)
    def _():
        pltpu.async_copy(g_hbm.at[pl.ds(s_off, RPS)],
                         g_sh.at[pl.ds(s_off, RPS)], tsem)

    @pl.when(s == NS - 1)
    def _():
        pltpu.async_copy(g_hbm.at[pl.ds(s_off, tail)],
                         g_sh.at[pl.ds(s_off, tail)], tsem)
        pltpu.async_copy(zeros_hbm.at[pl.ds(0, NPAD - N)],
                         g_sh.at[pl.ds(N, NPAD - N)], tsem)

    @pl.when((c == 0) & (s < NS - 1))
    def _():
        pltpu.async_copy(g_hbm.at[pl.ds(s_off, RPS)],
                         acc_sh.at[pl.ds(s_off, RPS)], asem)

    @pl.when((c == 0) & (s == NS - 1))
    def _():
        pltpu.async_copy(g_hbm.at[pl.ds(s_off, tail)],
                         acc_sh.at[pl.ds(s_off, tail)], asem)
        pltpu.async_copy(zeros_hbm.at[pl.ds(0, NPAD - N)],
                         acc_sh.at[pl.ds(N, NPAD - N)], asem)

    @pl.when(c == 1)
    def _():
        pltpu.async_copy(zeros_hbm.at[pl.ds(s_off, RPS)],
                         acc_sh.at[pl.ds(s_off, RPS)], asem)

    di1.wait()
    di2.wait()
    pltpu.make_async_copy(zeros_hbm.at[pl.ds(0, RPS)],
                          g_sh.at[pl.ds(s_off, RPS)], tsem).wait()
    pltpu.make_async_copy(zeros_hbm.at[pl.ds(0, RPS)],
                          acc_sh.at[pl.ds(s_off, RPS)], asem).wait()
    plsc.subcore_barrier()

    def gath(m, b):
        return pltpu.async_copy(g_sh.at[src_v.at[m]], rows_v[b], gsems[b])

    def scat(m, b):
        return pltpu.async_copy(rows_v[b], acc_sh.at[dst_v.at[m]],
                                ssems[b], add=True)

    gd = [None] * NSUP
    sd = [None] * NSUP
    for m in range(NSUP):
        b = m % 2
        if m >= 2:
            sd[m - 2].wait()
        gd[m] = gath(m, b)
        if m >= 1:
            gd[m - 1].wait()
            sd[m - 1] = scat(m - 1, (m - 1) % 2)
    gd[NSUP - 1].wait()
    sd[NSUP - 1] = scat(NSUP - 1, (NSUP - 1) % 2)
    for m in range(max(0, NSUP - 2), NSUP):
        sd[m].wait()

    plsc.subcore_barrier()
    pltpu.sync_copy(acc_sh.at[pl.ds(s * RPS, RPS)],
                    out_hbm.at[c].at[pl.ds(s * RPS, RPS)])


def _tc_stage1(x_ref, w_ref, deg_ref, g_ref, dis_ref):
    deg = (deg_ref[0] + deg_ref[1])[:N] + 1.0
    dis = lax.rsqrt(deg)
    h = jnp.dot(x_ref[...], w_ref[...], preferred_element_type=jnp.float32)
    g_ref[...] = h * dis
    dis_ref[...] = dis


def _tc_stage2(acc_ref, dis_ref, b_ref, w_ref, g2_ref):
    dis = dis_ref[...]
    srow = (acc_ref[0] + acc_ref[1])[:N]
    h = jnp.maximum(srow * dis + b_ref[...][None, :], 0.0)
    g2_ref[...] = jnp.dot(h, w_ref[...],
                          preferred_element_type=jnp.float32) * dis


def _tc_stage3(acc_ref, dis_ref, b_ref, out_ref):
    srow = (acc_ref[0] + acc_ref[1])[:N]
    out_ref[...] = srow * dis_ref[...] + b_ref[...][None, :]


def kernel(x, edge_index, W1, b1, W2, b2):
    ei = edge_index.astype(jnp.int32)
    pad = jnp.full((EPAD - E,), N, jnp.int32)
    src = jnp.concatenate([ei[0], pad]).reshape(NW * NSUP, SUP * CHUNK)
    dst = jnp.concatenate([ei[1], pad]).reshape(NW * NSUP, SUP * CHUNK)
    zeros_nd = jnp.zeros((NPAD, D_HID), jnp.float32)

    deg2 = _sc_degree(dst, zeros_nd)

    g1, dis = pl.pallas_call(
        _tc_stage1,
        out_shape=(jax.ShapeDtypeStruct((N, D_HID), jnp.float32),
                   jax.ShapeDtypeStruct((N, D_HID), jnp.float32)),
    )(x, W1, deg2)

    acc1 = _sc_edge_pass(g1, src, dst, zeros_nd)

    g2 = pl.pallas_call(
        _tc_stage2,
        out_shape=jax.ShapeDtypeStruct((N, D_HID), jnp.float32),
    )(acc1, dis, b1, W2)

    acc2 = _sc_edge_pass(g2, src, dst, zeros_nd)

    return pl.pallas_call(
        _tc_stage3,
        out_shape=jax.ShapeDtypeStruct((N, D_HID), jnp.float32),
    )(acc2, dis, b2)


# speedup vs baseline: 1.0292x; 1.0292x over previous
import functools

import jax
import jax.numpy as jnp
from jax import lax
from jax.experimental import pallas as pl
from jax.experimental.pallas import tpu as pltpu
from jax.experimental.pallas import tpu_sc as plsc

N = 10000
E = 320000
D_IN = 128
D_HID = 16

NC = 2
NS = 16
NW = NC * NS
CHUNK = 128
NCH = 80
Q = NCH * CHUNK
EPAD = NW * Q
NPAD = 10240
RPS = NPAD // NS
SUP = 8
NSUP = NCH // SUP

_mesh = plsc.VectorSubcoreMesh(core_axis_name="c", subcore_axis_name="s")
_sc_params = pltpu.CompilerParams(use_tc_tiling_on_sc=False,
                                  needs_layout_passes=False)


DROWS = 640


@functools.partial(
    pl.kernel,
    out_type=jax.ShapeDtypeStruct((NC, NPAD, 16), jnp.float32),
    mesh=_mesh,
    scratch_types=[
        pltpu.VMEM((NSUP, SUP * CHUNK), jnp.int32),
        pltpu.VMEM((DROWS, 16), jnp.float32),
        pltpu.VMEM((1, DROWS), jnp.int32),
        pltpu.VMEM((DROWS // NS, 16), jnp.float32),
        pltpu.VMEM((RPS, 16), jnp.float32),
        pltpu.VMEM_SHARED((DROWS, 16), jnp.float32),
        pltpu.SemaphoreType.DMA,
    ],
    compiler_params=_sc_params,
)
def _sc_degree(dst_hbm, zeros_hbm, out_hbm, idx_v, hist_v, iota_v, pk_v,
               bc_v, acc_sh, dsem):
    c = lax.axis_index("c")
    s = lax.axis_index("s")
    wid = c * NS + s
    di = pltpu.async_copy(dst_hbm.at[pl.ds(wid * NSUP, NSUP)], idx_v, dsem)
    zvec = jnp.zeros((16,), jnp.float32)

    @pl.loop(0, DROWS)
    def _(r):
        hist_v[r, :] = zvec

    @pl.loop(0, DROWS // 16)
    def _(m):
        iota_v[0, pl.ds(m * 16, 16)] = lax.iota(jnp.int32, 16) + m * 16

    @pl.when(s == 0)
    def _():
        pltpu.sync_copy(zeros_hbm.at[pl.ds(0, DROWS)], acc_sh)

    ones_vec = jnp.ones((16,), jnp.float32)
    di.wait()

    @pl.loop(0, NSUP)
    def _(j):
        for k in range(SUP * CHUNK // 16):
            d = idx_v[j, pl.ds(k * 16, 16)]
            plsc.addupdate_scatter(hist_v, [d >> 4, d & 15], ones_vec)

    plsc.subcore_barrier()
    pltpu.async_copy(hist_v, acc_sh.at[iota_v.at[0]], dsem, add=True).wait()
    plsc.subcore_barrier()
    pltpu.sync_copy(acc_sh.at[pl.ds(s * (DROWS // NS), DROWS // NS)], pk_v)

    @pl.loop(0, RPS)
    def _(i):
        bc_v[i, :] = plsc.load_gather(
            pk_v, [jnp.full((16,), i >> 4, jnp.int32),
                   jnp.full((16,), i & 15, jnp.int32)])

    pltpu.sync_copy(bc_v, out_hbm.at[c].at[pl.ds(s * RPS, RPS)])


@functools.partial(
    pl.kernel,
    out_type=jax.ShapeDtypeStruct((NC, NPAD, D_HID), jnp.float32),
    mesh=_mesh,
    scratch_types=[
        pltpu.VMEM((NSUP, SUP * CHUNK), jnp.int32),
        pltpu.VMEM((NSUP, SUP * CHUNK), jnp.int32),
        [pltpu.VMEM((SUP * CHUNK, D_HID), jnp.float32) for _ in range(4)],
        pltpu.VMEM_SHARED((NPAD, D_HID), jnp.float32),
        pltpu.VMEM_SHARED((NPAD, D_HID), jnp.float32),
        [pltpu.SemaphoreType.DMA for _ in range(4)],
        [pltpu.SemaphoreType.DMA for _ in range(4)],
        pltpu.SemaphoreType.DMA,
        pltpu.SemaphoreType.DMA,
        pltpu.SemaphoreType.DMA,
    ],
    compiler_params=_sc_params,
)
def _sc_edge_pass(g_hbm, src_hbm, dst_hbm, zeros_hbm, out_hbm,
                  src_v, dst_v, rows_v, acc_sh, g_sh, gsems, ssems,
                  isem, tsem, asem):
    c = lax.axis_index("c")
    s = lax.axis_index("s")
    wid = c * NS + s
    s_off = s * RPS
    tail = N - (NS - 1) * RPS
    di1 = pltpu.async_copy(src_hbm.at[pl.ds(wid * NSUP, NSUP)], src_v, isem)
    di2 = pltpu.async_copy(dst_hbm.at[pl.ds(wid * NSUP, NSUP)], dst_v, isem)

    @pl.when(s < NS - 1)
    def _():
        pltpu.async_copy(g_hbm.at[pl.ds(s_off, RPS)],
                         g_sh.at[pl.ds(s_off, RPS)], tsem)

    @pl.when(s == NS - 1)
    def _():
        pltpu.async_copy(g_hbm.at[pl.ds(s_off, tail)],
                         g_sh.at[pl.ds(s_off, tail)], tsem)
        pltpu.async_copy(zeros_hbm.at[pl.ds(0, NPAD - N)],
                         g_sh.at[pl.ds(N, NPAD - N)], tsem)

    @pl.when((c == 0) & (s < NS - 1))
    def _():
        pltpu.async_copy(g_hbm.at[pl.ds(s_off, RPS)],
                         acc_sh.at[pl.ds(s_off, RPS)], asem)

    @pl.when((c == 0) & (s == NS - 1))
    def _():
        pltpu.async_copy(g_hbm.at[pl.ds(s_off, tail)],
                         acc_sh.at[pl.ds(s_off, tail)], asem)
        pltpu.async_copy(zeros_hbm.at[pl.ds(0, NPAD - N)],
                         acc_sh.at[pl.ds(N, NPAD - N)], asem)

    @pl.when(c == 1)
    def _():
        pltpu.async_copy(zeros_hbm.at[pl.ds(s_off, RPS)],
                         acc_sh.at[pl.ds(s_off, RPS)], asem)

    di1.wait()
    di2.wait()
    pltpu.make_async_copy(zeros_hbm.at[pl.ds(0, RPS)],
                          g_sh.at[pl.ds(s_off, RPS)], tsem).wait()
    pltpu.make_async_copy(zeros_hbm.at[pl.ds(0, RPS)],
                          acc_sh.at[pl.ds(s_off, RPS)], asem).wait()
    plsc.subcore_barrier()

    def gath(m, b):
        return pltpu.async_copy(g_sh.at[src_v.at[m]], rows_v[b], gsems[b])

    def scat(m, b):
        return pltpu.async_copy(rows_v[b], acc_sh.at[dst_v.at[m]],
                                ssems[b], add=True)

    gd = [None] * NSUP
    sd = [None] * NSUP
    for m in range(NSUP):
        b = m % 4
        if m >= 4:
            sd[m - 4].wait()
        gd[m] = gath(m, b)
        if m >= 1:
            gd[m - 1].wait()
            sd[m - 1] = scat(m - 1, (m - 1) % 4)
    gd[NSUP - 1].wait()
    sd[NSUP - 1] = scat(NSUP - 1, (NSUP - 1) % 4)
    for m in range(NSUP - 4, NSUP):
        sd[m].wait()

    plsc.subcore_barrier()
    pltpu.sync_copy(acc_sh.at[pl.ds(s * RPS, RPS)],
                    out_hbm.at[c].at[pl.ds(s * RPS, RPS)])


def _tc_stage1(x_ref, w_ref, deg_ref, g_ref, dis_ref):
    deg = (deg_ref[0] + deg_ref[1])[:N] + 1.0
    dis = lax.rsqrt(deg)
    h = jnp.dot(x_ref[...], w_ref[...], preferred_element_type=jnp.float32)
    g_ref[...] = h * dis
    dis_ref[...] = dis


def _tc_stage2(acc_ref, dis_ref, b_ref, w_ref, g2_ref):
    dis = dis_ref[...]
    srow = (acc_ref[0] + acc_ref[1])[:N]
    h = jnp.maximum(srow * dis + b_ref[...][None, :], 0.0)
    g2_ref[...] = jnp.dot(h, w_ref[...],
                          preferred_element_type=jnp.float32) * dis


def _tc_stage3(acc_ref, dis_ref, b_ref, out_ref):
    srow = (acc_ref[0] + acc_ref[1])[:N]
    out_ref[...] = srow * dis_ref[...] + b_ref[...][None, :]


def kernel(x, edge_index, W1, b1, W2, b2):
    ei = edge_index.astype(jnp.int32)
    pad = jnp.full((EPAD - E,), N, jnp.int32)
    src = jnp.concatenate([ei[0], pad]).reshape(NW * NSUP, SUP * CHUNK)
    dst = jnp.concatenate([ei[1], pad]).reshape(NW * NSUP, SUP * CHUNK)
    zeros_nd = jnp.zeros((NPAD, D_HID), jnp.float32)

    deg2 = _sc_degree(dst, zeros_nd)

    g1, dis = pl.pallas_call(
        _tc_stage1,
        out_shape=(jax.ShapeDtypeStruct((N, D_HID), jnp.float32),
                   jax.ShapeDtypeStruct((N, D_HID), jnp.float32)),
    )(x, W1, deg2)

    acc1 = _sc_edge_pass(g1, src, dst, zeros_nd)

    g2 = pl.pallas_call(
        _tc_stage2,
        out_shape=jax.ShapeDtypeStruct((N, D_HID), jnp.float32),
    )(acc1, dis, b1, W2)

    acc2 = _sc_edge_pass(g2, src, dst, zeros_nd)

    return pl.pallas_call(
        _tc_stage3,
        out_shape=jax.ShapeDtypeStruct((N, D_HID), jnp.float32),
    )(acc2, dis, b2)
